# baseline (speedup 1.0000x reference)
import functools

import jax
import jax.numpy as jnp
from jax import lax
from jax.experimental import pallas as pl
from jax.experimental.pallas import tpu as pltpu
from jax.experimental.pallas import tpu_sc as plsc

NUM_EMBEDDINGS = 1000000
EMBEDDING_DIM = 32
THRESHOLD = 0.05

NC = 2
NS = 16
NW = NC * NS
L = 16

CB = 128
NBUF = 4
QBUF = 2

SB = 256
NFULL = NUM_EMBEDDINGS // SB
NTAIL = NUM_EMBEDDINGS - NFULL * SB


def _ternary(v):
    return jnp.where(
        v > THRESHOLD,
        jnp.float32(1.0),
        jnp.where(v < -THRESHOLD, jnp.float32(-1.0), jnp.float32(0.0)),
    )


def _ternary_fast(v):
    return jnp.where(jnp.abs(v) > THRESHOLD, jnp.sign(v), jnp.float32(0.0))


def _quantize_body(wt_hbm, wtail_hbm, q_hbm, band_v, tq_v, tail_v, *sems):
    gsems, osems = sems[:QBUF], sems[QBUF:]
    wid = lax.axis_index("s") * NC + lax.axis_index("c")
    lanes = lax.broadcasted_iota(jnp.int32, (L,), 0)

    nloc = NFULL // NW + 1

    def stage(slot, blk):
        co = blk * SB
        for b in range(EMBEDDING_DIM // 8):
            pltpu.async_copy(
                wt_hbm.at[pl.ds(b * 8, 8), pl.ds(co, SB)],
                band_v.at[slot, pl.ds(b * 8, 8)],
                gsems[slot],
            )

    def wait_stage(slot):
        for b in range(EMBEDDING_DIM // 8):
            pltpu.make_async_copy(
                wt_hbm.at[pl.ds(0, 8), pl.ds(0, SB)],
                band_v.at[slot, pl.ds(b * 8, 8)],
                gsems[slot],
            ).wait()

    def transpose_quantize(src, dst):
        @plsc.parallel_loop(0, L, unroll=2)
        def _(k):
            for half in range(EMBEDDING_DIM // L):
                c = ((lanes + k) & (L - 1)) + half * L

                @plsc.parallel_loop(0, SB // L)
                def _(h):
                    r = lanes + h * L
                    v = plsc.load_gather(src, [c, r])
                    plsc.store_scatter(dst, [r, c], _ternary_fast(v))

    for slot in range(QBUF):
        @pl.when(wid + slot * NW < NFULL)
        def _():
            stage(slot, wid + slot * NW)

    def loop(o, _):
        for slot in range(QBUF):
            i = o * QBUF + slot
            blk = wid + i * NW

            @pl.when(blk < NFULL)
            def _():
                wait_stage(slot)

                @pl.when(o > 0)
                def _():
                    pltpu.make_async_copy(
                        tq_v.at[slot], q_hbm.at[pl.ds(0, SB)], osems[slot]
                    ).wait()

                transpose_quantize(band_v.at[slot], tq_v.at[slot])
                pltpu.async_copy(
                    tq_v.at[slot], q_hbm.at[pl.ds(blk * SB, SB)], osems[slot]
                )

                nxt = blk + QBUF * NW

                @pl.when(nxt < NFULL)
                def _():
                    stage(slot, nxt)

        return 0

    lax.fori_loop(0, nloc // QBUF + 1, loop, 0)

    for slot in range(QBUF):
        @pl.when(wid + slot * NW < NFULL)
        def _():
            pltpu.make_async_copy(
                tq_v.at[slot], q_hbm.at[pl.ds(0, SB)], osems[slot]
            ).wait()

    @pl.when(wid == NW - 1)
    def _():
        pltpu.sync_copy(wtail_hbm, tail_v)
        for r in range(NTAIL):
            for h in range(EMBEDDING_DIM // L):
                tail_v[r, pl.ds(h * L, L)] = _ternary(
                    tail_v[r, pl.ds(h * L, L)]
                )
        pltpu.sync_copy(tail_v, q_hbm.at[pl.ds(NFULL * SB, NTAIL)])


def _transpose_chunk(rows, trans):
    lanes = lax.broadcasted_iota(jnp.int32, (L,), 0)

    @plsc.parallel_loop(0, L)
    def _(k):
        for half in range(EMBEDDING_DIM // L):
            c = ((lanes + k) & (L - 1)) + half * L
            i0 = c >> 3
            i1 = c & 7
            for h in range(CB // L):
                r = lanes + h * L
                v = plsc.load_gather(rows, [r, c])
                plsc.store_scatter(trans, [i0, i1, r], v)


def _gather_body(nchunk, ncb, table_hbm, idx_hbm, out_hbm, idx_v, rows_v,
                 trans_v, *sems):
    gsems, osems = sems[:NBUF], sems[NBUF:]
    wid = lax.axis_index("s") * NC + lax.axis_index("c")

    pltpu.sync_copy(idx_hbm.at[wid], idx_v)

    for b in range(NBUF):
        pltpu.async_copy(table_hbm.at[idx_v.at[b]], rows_v.at[b], gsems[b])

    nouter = nchunk // NBUF

    def outer(o, _):
        for b in range(NBUF):
            c = o * NBUF + b
            t = wid * nchunk + c
            j = t // ncb
            cb = lax.rem(t, ncb)
            buf = rows_v.at[b]
            tbuf = trans_v.at[b]
            pltpu.make_async_copy(
                table_hbm.at[idx_v.at[b]], buf, gsems[b]
            ).wait()

            @pl.when(o > 0)
            def _():
                pltpu.make_async_copy(
                    tbuf, out_hbm.at[0, :, 0, :, :], osems[b]
                ).wait()

            _transpose_chunk(buf, tbuf)
            pltpu.async_copy(tbuf, out_hbm.at[j, :, cb, :, :], osems[b])

            @pl.when(o < nouter - 1)
            def _():
                pltpu.async_copy(
                    table_hbm.at[idx_v.at[c + NBUF]], buf, gsems[b]
                )

        return 0

    lax.fori_loop(0, nouter, outer, 0)

    for b in range(NBUF):
        pltpu.make_async_copy(
            trans_v.at[b], out_hbm.at[0, :, 0, :, :], osems[b]
        ).wait()


def kernel(indices, weight):
    n, s = indices.shape
    b_total = n * s
    assert n % CB == 0 and b_total % (NW * CB * NBUF) == 0
    nchunk = b_total // (NW * CB)

    mesh = plsc.VectorSubcoreMesh(
        core_axis_name="c", subcore_axis_name="s", num_cores=NC, num_subcores=NS
    )

    quantize = pl.kernel(
        _quantize_body,
        out_type=jax.ShapeDtypeStruct((NUM_EMBEDDINGS, EMBEDDING_DIM),
                                      jnp.float32),
        mesh=mesh,
        scratch_types=[
            pltpu.VMEM((QBUF, EMBEDDING_DIM, SB), jnp.float32),
            pltpu.VMEM((QBUF, SB, EMBEDDING_DIM), jnp.float32),
            pltpu.VMEM((NTAIL, EMBEDDING_DIM), jnp.float32),
        ]
        + [pltpu.SemaphoreType.DMA] * (2 * QBUF),
        compiler_params=pltpu.CompilerParams(
            use_tc_tiling_on_sc=True, needs_layout_passes=False
        ),
    )
    q_table = quantize(weight.T, weight[NFULL * SB:])

    idx3d = indices.T.reshape(NW, nchunk, CB)

    gather = pl.kernel(
        functools.partial(_gather_body, nchunk, n // CB),
        out_type=jax.ShapeDtypeStruct(
            (s, EMBEDDING_DIM // 8, n // CB, 8, CB), jnp.float32
        ),
        mesh=mesh,
        scratch_types=[
            pltpu.VMEM((nchunk, CB), jnp.int32),
            pltpu.VMEM((NBUF, CB, EMBEDDING_DIM), jnp.float32),
            pltpu.VMEM((NBUF, EMBEDDING_DIM // 8, 8, CB), jnp.float32),
        ]
        + [pltpu.SemaphoreType.DMA] * (2 * NBUF),
        compiler_params=pltpu.CompilerParams(
            use_tc_tiling_on_sc=False, needs_layout_passes=False
        ),
    )
    out5d = gather(q_table, idx3d)
    return out5d.transpose(2, 4, 0, 1, 3).reshape(n, s, EMBEDDING_DIM)


# speedup vs baseline: 1.1904x; 1.1904x over previous
import functools

import jax
import jax.numpy as jnp
from jax import lax
from jax.experimental import pallas as pl
from jax.experimental.pallas import tpu as pltpu
from jax.experimental.pallas import tpu_sc as plsc

NUM_EMBEDDINGS = 1000000
EMBEDDING_DIM = 32
THRESHOLD = 0.05

NC = 2
NS = 16
NW = NC * NS
L = 16

CB = 128
NBUF = 4


def _ternary(v):
    return jnp.where(
        v > THRESHOLD,
        jnp.float32(1.0),
        jnp.where(v < -THRESHOLD, jnp.float32(-1.0), jnp.float32(0.0)),
    )


def _quantize_transpose(rows, trans):
    lanes = lax.broadcasted_iota(jnp.int32, (L,), 0)

    @plsc.parallel_loop(0, L)
    def _(k):
        for half in range(EMBEDDING_DIM // L):
            c = ((lanes + k) & (L - 1)) + half * L
            i0 = c >> 3
            i1 = c & 7
            for h in range(CB // L):
                r = lanes + h * L
                v = plsc.load_gather(rows, [r, c])
                plsc.store_scatter(trans, [i0, i1, r], _ternary(v))


def _sc_body(nchunk, ncb, table_hbm, idx_hbm, out_hbm, idx_v, rows_v, trans_v, *sems):
    gsems, osems = sems[:NBUF], sems[NBUF:]
    wid = lax.axis_index("s") * NC + lax.axis_index("c")

    pltpu.sync_copy(idx_hbm.at[wid], idx_v)

    for b in range(NBUF):
        pltpu.async_copy(table_hbm.at[idx_v.at[b]], rows_v.at[b], gsems[b])

    nouter = nchunk // NBUF

    def outer(o, _):
        for b in range(NBUF):
            c = o * NBUF + b
            t = wid * nchunk + c
            j = t // ncb
            cb = lax.rem(t, ncb)
            buf = rows_v.at[b]
            tbuf = trans_v.at[b]
            pltpu.make_async_copy(
                table_hbm.at[idx_v.at[b]], buf, gsems[b]
            ).wait()

            @pl.when(o > 0)
            def _():
                pltpu.make_async_copy(
                    tbuf, out_hbm.at[0, :, 0, :, :], osems[b]
                ).wait()

            _quantize_transpose(buf, tbuf)
            pltpu.async_copy(tbuf, out_hbm.at[j, :, cb, :, :], osems[b])

            @pl.when(o < nouter - 1)
            def _():
                pltpu.async_copy(
                    table_hbm.at[idx_v.at[c + NBUF]], buf, gsems[b]
                )

        return 0

    lax.fori_loop(0, nouter, outer, 0)

    for b in range(NBUF):
        pltpu.make_async_copy(
            trans_v.at[b], out_hbm.at[0, :, 0, :, :], osems[b]
        ).wait()


def kernel(indices, weight):
    n, s = indices.shape
    b_total = n * s
    assert n % CB == 0 and b_total % (NW * CB * NBUF) == 0
    nchunk = b_total // (NW * CB)

    idx3d = indices.T.reshape(NW, nchunk, CB)

    mesh = plsc.VectorSubcoreMesh(
        core_axis_name="c", subcore_axis_name="s", num_cores=NC, num_subcores=NS
    )
    run = pl.kernel(
        functools.partial(_sc_body, nchunk, n // CB),
        out_type=jax.ShapeDtypeStruct(
            (s, EMBEDDING_DIM // 8, n // CB, 8, CB), jnp.float32
        ),
        mesh=mesh,
        scratch_types=[
            pltpu.VMEM((nchunk, CB), jnp.int32),
            pltpu.VMEM((NBUF, CB, EMBEDDING_DIM), jnp.float32),
            pltpu.VMEM((NBUF, EMBEDDING_DIM // 8, 8, CB), jnp.float32),
        ]
        + [pltpu.SemaphoreType.DMA] * (2 * NBUF),
        compiler_params=pltpu.CompilerParams(
            use_tc_tiling_on_sc=False, needs_layout_passes=False
        ),
    )
    out5d = run(weight, idx3d)
    return out5d.transpose(2, 4, 0, 1, 3).reshape(n, s, EMBEDDING_DIM)
